# Initial kernel scaffold; baseline (speedup 1.0000x reference)
import jax
import jax.numpy as jnp
from jax.experimental import pallas as pl


def kernel(y, charge, mz, emb_w, charge_w, ln1_g, ln1_b, ln2_g, ln2_b, pe_table, mz_div):
    raise NotImplementedError("write your pallas kernel here")


# speedup vs baseline: 6.4652x; 6.4652x over previous
import jax
import jax.numpy as jnp
from jax import lax
from jax.experimental import pallas as pl

_L = 50
_D = 128
_V = 32
_CPAD = 16


def _body(y_ref, ch_ref, mz_ref, pe_ref, emb_ref, chw_ref,
          g1_ref, b1_ref, g2_ref, b2_ref, mzd_ref, out_ref):
    rows = out_ref.shape[0]
    bb = ch_ref.shape[-1]

    t = pe_ref[...] + emb_ref[...]
    mu = jnp.mean(t, axis=-1, keepdims=True)
    var = jnp.mean((t - mu) * (t - mu), axis=-1, keepdims=True)
    fused = (t - mu) / jnp.sqrt(var + 1e-5) * g1_ref[...] + b1_ref[...]

    ch = ch_ref[0, 0, :]
    oc = (ch[:, None] == lax.broadcasted_iota(jnp.int32, (bb, _CPAD), 1))
    cemb = jnp.dot(oc.astype(jnp.float32), chw_ref[...],
                   preferred_element_type=jnp.float32)
    inp = jnp.floor(mz_ref[0, 0, :] / 0.001)
    arg = inp[:, None] * mzd_ref[...]
    par = lax.broadcasted_iota(jnp.int32, (bb, _D), 1) % 2
    mzpe = jnp.where(par == 0, jnp.sin(arg), jnp.cos(arg))
    bits = lax.bitcast_convert_type(mzpe, jnp.int32)
    bits = bits + 0x0FFF + ((bits >> 13) & 1)
    mzpe = lax.bitcast_convert_type(bits & jnp.int32(-8192), jnp.float32)
    extra = cemb + mzpe

    tok = y_ref[0, 0, :]
    ot = (tok[:, None] == lax.broadcasted_iota(jnp.int32, (rows, _V), 1))
    pep = jnp.dot(ot.astype(jnp.float32), fused,
                  preferred_element_type=jnp.float32)
    rb = lax.broadcasted_iota(jnp.int32, (rows, bb), 0) // _L
    ob = (rb == lax.broadcasted_iota(jnp.int32, (rows, bb), 1))
    ext = jnp.dot(ob.astype(jnp.float32), extra,
                  preferred_element_type=jnp.float32)

    x = pep + ext
    m2 = jnp.mean(x, axis=-1, keepdims=True)
    v2 = jnp.mean((x - m2) * (x - m2), axis=-1, keepdims=True)
    out_ref[...] = (x - m2) / jnp.sqrt(v2 + 1e-5) * g2_ref[...] + b2_ref[...]


def kernel(y, charge, mz, emb_w, charge_w, ln1_g, ln1_b, ln2_g, ln2_b,
           pe_table, mz_div):
    B, L = y.shape
    D = emb_w.shape[1]
    BB = 64
    ROWS = BB * L
    grid = B // BB

    y3 = y.reshape(grid, 1, ROWS)
    ch3 = charge.astype(jnp.int32).reshape(grid, 1, BB)
    mz3 = mz.reshape(grid, 1, BB)
    pe32 = pe_table[:_V]
    chw = jnp.zeros((_CPAD, D), jnp.float32).at[:charge_w.shape[0]].set(charge_w)
    mzd = jnp.repeat(mz_div, 2).reshape(1, D)

    out2 = pl.pallas_call(
        _body,
        grid=(grid,),
        in_specs=[
            pl.BlockSpec((1, 1, ROWS), lambda i: (i, 0, 0)),
            pl.BlockSpec((1, 1, BB), lambda i: (i, 0, 0)),
            pl.BlockSpec((1, 1, BB), lambda i: (i, 0, 0)),
            pl.BlockSpec((_V, D), lambda i: (0, 0)),
            pl.BlockSpec((_V, D), lambda i: (0, 0)),
            pl.BlockSpec((_CPAD, D), lambda i: (0, 0)),
            pl.BlockSpec((1, D), lambda i: (0, 0)),
            pl.BlockSpec((1, D), lambda i: (0, 0)),
            pl.BlockSpec((1, D), lambda i: (0, 0)),
            pl.BlockSpec((1, D), lambda i: (0, 0)),
            pl.BlockSpec((1, D), lambda i: (0, 0)),
        ],
        out_specs=pl.BlockSpec((ROWS, D), lambda i: (i, 0)),
        out_shape=jax.ShapeDtypeStruct((B * L, D), jnp.float32),
    )(y3, ch3, mz3, pe32, emb_w, chw,
      ln1_g.reshape(1, D), ln1_b.reshape(1, D),
      ln2_g.reshape(1, D), ln2_b.reshape(1, D), mzd)
    return out2.reshape(B, L, D)


# speedup vs baseline: 6.7921x; 1.0506x over previous
import jax
import jax.numpy as jnp
from jax import lax
from jax.experimental import pallas as pl

_L = 50
_D = 128
_V = 32
_CPAD = 16


def _body(y_ref, ch_ref, mz_ref, pe_ref, emb_ref, chw_ref,
          g1_ref, b1_ref, g2_ref, b2_ref, mzd_ref, ob_ref, out_ref):
    rows = out_ref.shape[0]
    bb = ch_ref.shape[-1]

    t = pe_ref[...] + emb_ref[...]
    mu = jnp.mean(t, axis=-1, keepdims=True)
    var = jnp.mean((t - mu) * (t - mu), axis=-1, keepdims=True)
    fused = (t - mu) / jnp.sqrt(var + 1e-5) * g1_ref[...] + b1_ref[...]

    ch = ch_ref[0, 0, :]
    oc = (ch[:, None] == lax.broadcasted_iota(jnp.int32, (bb, _CPAD), 1))
    cemb = jnp.dot(oc.astype(jnp.float32), chw_ref[...],
                   preferred_element_type=jnp.float32)
    inp = jnp.floor(mz_ref[0, 0, :] / 0.001)
    arg = inp[:, None] * mzd_ref[...]
    par = lax.broadcasted_iota(jnp.int32, (bb, _D), 1) % 2
    mzpe = jnp.where(par == 0, jnp.sin(arg), jnp.cos(arg))
    bits = lax.bitcast_convert_type(mzpe, jnp.int32)
    bits = bits + 0x0FFF + ((bits >> 13) & 1)
    mzpe = lax.bitcast_convert_type(bits & jnp.int32(-8192), jnp.float32)
    extra = cemb + mzpe

    tok = y_ref[0, 0, :]
    ot = (tok[:, None] == lax.broadcasted_iota(jnp.int32, (rows, _V), 1))
    pep = jnp.dot(ot.astype(jnp.float32), fused,
                  preferred_element_type=jnp.float32)
    ext = jnp.dot(ob_ref[...], extra, preferred_element_type=jnp.float32)

    x = pep + ext
    m2 = jnp.mean(x, axis=-1, keepdims=True)
    msq = jnp.mean(x * x, axis=-1, keepdims=True)
    rs = lax.rsqrt(msq - m2 * m2 + 1e-5)
    out_ref[...] = (x - m2) * rs * g2_ref[...] + b2_ref[...]


def kernel(y, charge, mz, emb_w, charge_w, ln1_g, ln1_b, ln2_g, ln2_b,
           pe_table, mz_div):
    B, L = y.shape
    D = emb_w.shape[1]
    BB = 128
    ROWS = BB * L
    grid = B // BB

    y3 = y.reshape(grid, 1, ROWS)
    ch3 = charge.astype(jnp.int32).reshape(grid, 1, BB)
    mz3 = mz.reshape(grid, 1, BB)
    pe32 = pe_table[:_V]
    chw = jnp.zeros((_CPAD, D), jnp.float32).at[:charge_w.shape[0]].set(charge_w)
    mzd = jnp.repeat(mz_div, 2).reshape(1, D)
    ob = jnp.repeat(jnp.eye(BB, dtype=jnp.float32), L, axis=0)

    out2 = pl.pallas_call(
        _body,
        grid=(grid,),
        in_specs=[
            pl.BlockSpec((1, 1, ROWS), lambda i: (i, 0, 0)),
            pl.BlockSpec((1, 1, BB), lambda i: (i, 0, 0)),
            pl.BlockSpec((1, 1, BB), lambda i: (i, 0, 0)),
            pl.BlockSpec((_V, D), lambda i: (0, 0)),
            pl.BlockSpec((_V, D), lambda i: (0, 0)),
            pl.BlockSpec((_CPAD, D), lambda i: (0, 0)),
            pl.BlockSpec((1, D), lambda i: (0, 0)),
            pl.BlockSpec((1, D), lambda i: (0, 0)),
            pl.BlockSpec((1, D), lambda i: (0, 0)),
            pl.BlockSpec((1, D), lambda i: (0, 0)),
            pl.BlockSpec((1, D), lambda i: (0, 0)),
            pl.BlockSpec((ROWS, BB), lambda i: (0, 0)),
        ],
        out_specs=pl.BlockSpec((ROWS, D), lambda i: (i, 0)),
        out_shape=jax.ShapeDtypeStruct((B * L, D), jnp.float32),
    )(y3, ch3, mz3, pe32, emb_w, chw,
      ln1_g.reshape(1, D), ln1_b.reshape(1, D),
      ln2_g.reshape(1, D), ln2_b.reshape(1, D), mzd, ob)
    return out2.reshape(B, L, D)
